# Initial kernel scaffold; baseline (speedup 1.0000x reference)
import jax
import jax.numpy as jnp
from jax.experimental import pallas as pl


def kernel(sp_feat, rawPoint_feat, point_assignments, Wq, Wkv, ln_gamma, ln_beta):
    raise NotImplementedError("write your pallas kernel here")


# speedup vs baseline: 16.5208x; 16.5208x over previous
import functools

import jax
import jax.numpy as jnp
from jax import lax
from jax.experimental import pallas as pl
from jax.experimental.pallas import tpu as pltpu
from jax.experimental.pallas import tpu_sc as plsc

NUM_SP = 4096
EMB = 256
H = 8
DEPTH = EMB // H
SCALE = DEPTH ** (-0.5)

NC = 2
NS = 16
NW = NC * NS
C = 112
BLK = 512

F32 = jnp.float32
I32 = jnp.int32


def _prep_body(sp_ref, wq_ref, wkv_ref, q_ref, wk_ref, wv_ref):
    q_ref[...] = jnp.dot(sp_ref[...], wq_ref[...],
                         preferred_element_type=F32)
    wk_ref[...] = wkv_ref[0:EMB, 0:EMB] + wkv_ref[EMB:2 * EMB, 0:EMB]
    wv_ref[...] = (wkv_ref[0:EMB, EMB:2 * EMB]
                   + wkv_ref[EMB:2 * EMB, EMB:2 * EMB])


def _prep(sp_feat, wq, wkv):
    return pl.pallas_call(
        _prep_body,
        out_shape=(
            jax.ShapeDtypeStruct((NUM_SP, EMB), F32),
            jax.ShapeDtypeStruct((EMB, EMB), F32),
            jax.ShapeDtypeStruct((EMB, EMB), F32),
        ),
    )(sp_feat, wq, wkv)


def _gather_q(q, a_pad, npad, pw):
    mesh = plsc.VectorSubcoreMesh(core_axis_name="c", subcore_axis_name="s")

    @functools.partial(
        pl.kernel,
        out_type=jax.ShapeDtypeStruct((npad, EMB), F32),
        mesh=mesh,
        scratch_types=[
            pltpu.VMEM((C,), I32),
            pltpu.VMEM((C, EMB), F32),
            pltpu.SemaphoreType.DMA,
        ],
    )
    def k(q_hbm, a_hbm, qg_hbm, idx_v, rows_v, sem):
        wid = lax.axis_index("s") * NC + lax.axis_index("c")
        base = wid * pw

        def body(j, carry):
            off = base + j * C
            pltpu.sync_copy(a_hbm.at[pl.ds(off, C)], idx_v)
            pltpu.async_copy(q_hbm.at[idx_v], rows_v, sem).wait()
            pltpu.sync_copy(rows_v, qg_hbm.at[pl.ds(off, C)])
            return carry

        lax.fori_loop(0, pw // C, body, 0)

    return k(q, a_pad)


def _kv_body(n, raw_ref, qg_ref, wk_ref, wv_ref, r_ref,
             v_ref, s_ref, st_ref, attn_ref, gmax_ref):
    i = pl.program_id(0)
    raw = raw_ref[...]
    kmat = jnp.dot(raw, wk_ref[...], preferred_element_type=F32)
    p = qg_ref[...] * kmat
    s = jnp.dot(p, r_ref[...], preferred_element_type=F32) * SCALE
    rowid = i * BLK + lax.broadcasted_iota(I32, (BLK, 1), 0)
    m = rowid < n
    s = jnp.where(m, s, 0.0)
    vmat = jnp.dot(raw, wv_ref[...], preferred_element_type=F32)
    v_ref[...] = jnp.where(m, vmat, 0.0)
    s_ref[...] = jnp.concatenate([s, jnp.zeros((BLK, H), F32)], axis=1)
    st = lax.dot_general(r_ref[...], p,
                         (((0,), (1,)), ((), ())),
                         preferred_element_type=F32) * SCALE
    colid = i * BLK + lax.broadcasted_iota(I32, (H, BLK), 1)
    st_ref[...] = jnp.where(colid < n, st, 0.0)
    attn_ref[...] = jnp.sum(s, axis=1, keepdims=True) * (1.0 / H)
    bm = jnp.max(s, keepdims=True).reshape(1, 1)
    prev = jnp.where(i == 0, jnp.full((1, 1), -1e30, F32), gmax_ref[...])
    gmax_ref[...] = jnp.maximum(prev, bm)


def _kv_scores(raw, qg, wk, wv, rmat, n, npad):
    grid = npad // BLK
    return pl.pallas_call(
        functools.partial(_kv_body, n),
        grid=(grid,),
        in_specs=[
            pl.BlockSpec((BLK, EMB), lambda i: (i, 0)),
            pl.BlockSpec((BLK, EMB), lambda i: (i, 0)),
            pl.BlockSpec((EMB, EMB), lambda i: (0, 0)),
            pl.BlockSpec((EMB, EMB), lambda i: (0, 0)),
            pl.BlockSpec((EMB, H), lambda i: (0, 0)),
        ],
        out_specs=[
            pl.BlockSpec((BLK, EMB), lambda i: (i, 0)),
            pl.BlockSpec((BLK, 2 * H), lambda i: (i, 0)),
            pl.BlockSpec((H, BLK), lambda i: (0, i)),
            pl.BlockSpec((BLK, 1), lambda i: (i, 0)),
            pl.BlockSpec((1, 1), lambda i: (0, 0)),
        ],
        out_shape=[
            jax.ShapeDtypeStruct((npad, EMB), F32),
            jax.ShapeDtypeStruct((npad, 2 * H), F32),
            jax.ShapeDtypeStruct((H, npad), F32),
            jax.ShapeDtypeStruct((npad, 1), F32),
            jax.ShapeDtypeStruct((1, 1), F32),
        ],
    )(raw, qg, wk, wv, rmat)


C1 = 784
CC = 1792


def _scatter(v, s16, st, a_pad, g16, n, npad, pw):
    mesh = plsc.VectorSubcoreMesh(core_axis_name="c", subcore_axis_name="s")
    nhalf = npad // 2

    @functools.partial(
        pl.kernel,
        out_type=(
            jax.ShapeDtypeStruct((NC * NS, NUM_SP, 16), F32),
            jax.ShapeDtypeStruct((NC * NS, NUM_SP, 16), F32),
        ),
        mesh=mesh,
        scratch_types=[
            pltpu.VMEM((NUM_SP, 16), F32),
            pltpu.VMEM((CC, 16), F32),
            pltpu.VMEM((C1, 16), F32),
            pltpu.VMEM((CC,), F32),
            pltpu.VMEM((CC,), I32),
            pltpu.VMEM((16,), F32),
        ],
        compiler_params=pltpu.CompilerParams(use_tc_tiling_on_sc=False),
    )
    def k(v_hbm, s_hbm, st_hbm, a_hbm, g_hbm, ctx_hbm, l_hbm,
          acc, vslice, sbuf, ebuf, abuf, gbuf):
        cid = lax.axis_index("c")
        sid = lax.axis_index("s")
        wid = sid * NC + cid
        pltpu.sync_copy(g_hbm, gbuf)
        gv = gbuf[...]
        lane = lax.iota(I32, 16)
        zero16 = jnp.zeros((16,), F32)

        def zrow(r, carry):
            acc[r, :] = zero16
            return carry

        lax.fori_loop(0, NUM_SP, zrow, 0)
        base = wid * pw

        def ph1(u, carry):
            off = base + u * C1
            pltpu.sync_copy(s_hbm.at[pl.ds(off, C1)], sbuf)
            pltpu.sync_copy(a_hbm.at[pl.ds(off, C1)], abuf.at[pl.ds(0, C1)])

            def grp(g, carry2):
                avec = abuf[pl.ds(g * 16, 16)]
                for p in range(16):
                    idxp = g * 16 + p
                    srow = sbuf[idxp, :]
                    ev = jnp.where(lane < H, jnp.exp(srow - gv), 0.0)
                    ev = ev * jnp.where(off + idxp < n, 1.0, 0.0)
                    plsc.addupdate(acc.at[avec[p]], ev)
                return carry2

            lax.fori_loop(0, C1 // 16, grp, 0)
            return carry

        lax.fori_loop(0, pw // C1, ph1, 0)
        pltpu.sync_copy(acc, l_hbm.at[wid])

        lax.fori_loop(0, NUM_SP, zrow, 0)
        hid = sid // 2
        cbase = cid * nhalf

        def ph2(t, carry):
            off = cbase + t * CC
            pltpu.sync_copy(v_hbm.at[pl.ds(off, CC), pl.ds(sid * 16, 16)],
                            vslice)
            pltpu.sync_copy(st_hbm.at[hid, pl.ds(off, CC)], ebuf)
            pltpu.sync_copy(a_hbm.at[pl.ds(off, CC)], abuf)

            def grp(g, carry2):
                b16 = g * 16
                ev = jnp.exp(ebuf[pl.ds(b16, 16)] - gv)
                ev = jnp.where(off + b16 + lane < n, ev, 0.0)
                avec = abuf[pl.ds(b16, 16)]
                for p in range(16):
                    row = vslice[b16 + p, :]
                    plsc.addupdate(acc.at[avec[p]], row * ev[p])
                return carry2

            lax.fori_loop(0, CC // 16, grp, 0)
            return carry

        lax.fori_loop(0, nhalf // CC, ph2, 0)
        pltpu.sync_copy(acc, ctx_hbm.at[wid])

    return k(v, s16, st, a_pad, g16)


def _fin_body(sp_ref, ctx_ref, l_ref, e_ref, g_ref, b_ref, out_ref):
    l8 = jnp.sum(l_ref[...], axis=0)[:, 0:H]
    inv = jnp.where(l8 > 0, 1.0 / l8, 0.0)
    invf = jnp.dot(inv, e_ref[...], preferred_element_type=F32)
    ctx = jnp.concatenate(
        [ctx_ref[2 * t] + ctx_ref[2 * t + 1] for t in range(NS)], axis=1)
    x = sp_ref[...] + ctx * invf
    mu = jnp.mean(x, axis=1, keepdims=True)
    var = jnp.mean((x - mu) ** 2, axis=1, keepdims=True)
    out_ref[...] = (x - mu) / jnp.sqrt(var + 1e-5) * g_ref[...] + b_ref[...]


def _finish(sp_feat, ctx2, l2, emat, gamma, beta):
    rb = 512
    grid = NUM_SP // rb
    return pl.pallas_call(
        _fin_body,
        grid=(grid,),
        in_specs=[
            pl.BlockSpec((rb, EMB), lambda i: (i, 0)),
            pl.BlockSpec((NC * NS, rb, 16), lambda i: (0, i, 0)),
            pl.BlockSpec((NC * NS, rb, 16), lambda i: (0, i, 0)),
            pl.BlockSpec((H, EMB), lambda i: (0, 0)),
            pl.BlockSpec((1, EMB), lambda i: (0, 0)),
            pl.BlockSpec((1, EMB), lambda i: (0, 0)),
        ],
        out_specs=pl.BlockSpec((rb, EMB), lambda i: (i, 0)),
        out_shape=jax.ShapeDtypeStruct((NUM_SP, EMB), F32),
    )(sp_feat, ctx2, l2, emat, gamma, beta)


def kernel(sp_feat, rawPoint_feat, point_assignments, Wq, Wkv,
           ln_gamma, ln_beta):
    n = rawPoint_feat.shape[0]
    chunks_per_worker = -(-n // (NW * C))
    pw = chunks_per_worker * C
    npad = NW * pw
    assert npad % BLK == 0

    a_pad = jnp.pad(point_assignments.astype(I32), (0, npad - n))
    rmat = jnp.kron(jnp.eye(H, dtype=F32), jnp.ones((DEPTH, 1), F32))
    emat = jnp.kron(jnp.eye(H, dtype=F32), jnp.ones((1, DEPTH), F32))

    q, wk, wv = _prep(sp_feat, Wq, Wkv)
    qg = _gather_q(q, a_pad, npad, pw)
    v, s16, st, attn, gmax = _kv_scores(rawPoint_feat, qg, wk, wv, rmat,
                                        n, npad)
    g16 = jnp.broadcast_to(gmax[0, 0], (16,))
    ctx2, l2 = _scatter(v, s16, st, a_pad, g16, n, npad, pw)
    updated = _finish(sp_feat, ctx2, l2, emat,
                      ln_gamma.reshape(1, EMB), ln_beta.reshape(1, EMB))
    attn_scores = attn[:n, 0]
    return (updated, attn_scores)
